# baseline (speedup 1.0000x reference)
import functools

import jax
import jax.numpy as jnp
from jax import lax
from jax.experimental import pallas as pl
from jax.experimental.pallas import tpu as pltpu
from jax.experimental.pallas import tpu_sc as plsc

D_MODEL = 384
N_HEADS = 8
N_POINTS = 4
PATCH = 16
C_IN = 3
HF = 32
LQ = HF * HF
DH = D_MODEL // N_HEADS
MP = N_HEADS * N_POINTS


def _patchify_body(x_hbm, p_hbm, pbuf, sem, wsem):
    wid = lax.axis_index("s") * 2 + lax.axis_index("c")
    n = wid // 4
    qb = wid % 4

    wcps = [None, None]
    for rr in range(4):
        qy0 = qb * 8 + rr * 2
        pp = rr % 2
        if wcps[pp] is not None:
            wcps[pp].wait()
        copies = []
        for j in range(48):
            c, py = divmod(j, PATCH)
            i, sub = divmod(j, 8)
            copies.append(pltpu.async_copy(
                x_hbm.at[n, c, pl.ds(qy0, 2), py, :, :],
                pbuf.at[pp, i, :, :, pl.ds(sub * 16, 16)], sem))
        for cp in copies:
            cp.wait()
        wcps[pp] = pltpu.async_copy(
            pbuf.at[pp],
            p_hbm.at[n, :, qb * 2 + rr // 2, pl.ds((rr % 2) * 2, 2)], wsem)
    for cp in wcps:
        cp.wait()


def _dott(a, b):
    return lax.dot_general(a, b, (((1,), (1,)), ((), ())),
                           preferred_element_type=jnp.float32)


def _embed_body(p_ref, we_ref, be_ref, wv_ref, bv_ref, wso_ref, bso_ref,
                waw_ref, baw_ref, val_ref, sidx_ref, scoef_ref):
    p6 = p_ref[0].astype(jnp.bfloat16)
    feat = be_ref[...]
    for i in range(6):
        feat = feat + _dott(p6[i], we_ref[:, i, :])
    fb = feat.astype(jnp.bfloat16)
    val_ref[0] = _dott(fb, wv_ref[...]) + bv_ref[...]
    so = _dott(fb, wso_ref[...]) + bso_ref[...]
    awl = _dott(fb, waw_ref[...]) + baw_ref[...]

    rowmax = jnp.max(awl, axis=1, keepdims=True)
    e = jnp.exp(awl - rowmax)
    gi = lax.broadcasted_iota(jnp.int32, (MP, MP), 0) // N_POINTS
    gj = lax.broadcasted_iota(jnp.int32, (MP, MP), 1) // N_POINTS
    smat = (gi == gj).astype(jnp.float32)
    aw = e / jnp.dot(e, smat, preferred_element_type=jnp.float32)

    q = lax.broadcasted_iota(jnp.int32, (LQ, 1), 0)
    qx = (q % HF).astype(jnp.float32)
    qy = (q // HF).astype(jnp.float32)
    sc = jnp.float32(HF / (HF - 1.0))
    gx = qx * sc - 0.5 + so[:, :MP]
    gy = qy * sc - 0.5 + so[:, MP:]

    x0 = jnp.floor(gx)
    y0 = jnp.floor(gy)
    wx1 = gx - x0
    wx0 = 1.0 - wx1
    wy1 = gy - y0
    wy0 = 1.0 - wy1
    f32 = jnp.float32
    vx0 = ((x0 >= 0) & (x0 <= HF - 1)).astype(f32)
    vx1 = ((x0 >= -1) & (x0 <= HF - 2)).astype(f32)
    vy0 = ((y0 >= 0) & (y0 <= HF - 1)).astype(f32)
    vy1 = ((y0 >= -1) & (y0 <= HF - 2)).astype(f32)
    ax0 = wx0 * vx0
    ax1 = wx1 * vx1
    t0 = aw * (wy0 * vy0)
    t1 = aw * (wy1 * vy1)
    x0c = jnp.clip(x0, 0.0, HF - 1.0).astype(jnp.int32)
    x1c = jnp.clip(x0 + 1.0, 0.0, HF - 1.0).astype(jnp.int32)
    y0c = jnp.clip(y0, 0.0, HF - 1.0).astype(jnp.int32)
    y1c = jnp.clip(y0 + 1.0, 0.0, HF - 1.0).astype(jnp.int32)
    x128 = jnp.concatenate([x0c, x1c, x0c, x1c], axis=1)
    y128 = jnp.concatenate([y0c, y0c, y1c, y1c], axis=1)
    t128 = jnp.concatenate([t0, t0, t1, t1], axis=1)
    ax128 = jnp.concatenate([ax0, ax1, ax0, ax1], axis=1)
    sidx_ref[0] = y128 * HF + x128
    scoef_ref[0] = t128 * ax128


def _contract_body(coef_ref, val_ref, opw_ref, opb_ref, prw_ref, prb_ref, out_ref):
    c4 = coef_ref[0]
    csum = (c4[0] + c4[1]) + (c4[2] + c4[3])
    v3 = val_ref[0].reshape(N_HEADS, 128, D_MODEL)
    r = jnp.zeros((N_HEADS, D_MODEL), jnp.float32)
    for g in range(8):
        r = r + jnp.dot(csum[g * 8:(g + 1) * 8], v3[g],
                        preferred_element_type=jnp.float32)
    col = lax.broadcasted_iota(jnp.int32, (N_HEADS, D_MODEL), 1) // DH
    row = lax.broadcasted_iota(jnp.int32, (N_HEADS, D_MODEL), 0)
    mask = (col == row).astype(jnp.float32)
    osum = jnp.sum(r * mask, axis=0, keepdims=True) * jnp.float32(1.0 / LQ)
    h = _dott(osum, opw_ref[...]) + opb_ref[...]
    out_ref[0] = _dott(h, prw_ref[...]) + prb_ref[...]


_sc_patchify = functools.partial(
    pl.kernel,
    out_type=jax.ShapeDtypeStruct((8, 6, 8, 4, HF, 128), jnp.float32),
    mesh=plsc.VectorSubcoreMesh(core_axis_name="c", subcore_axis_name="s",
                                num_cores=2, num_subcores=16),
    compiler_params=pltpu.CompilerParams(use_tc_tiling_on_sc=False,
                                         needs_layout_passes=False),
    scratch_types=[
        pltpu.VMEM((2, 6, 2, HF, 128), jnp.float32),
        pltpu.SemaphoreType.DMA,
        pltpu.SemaphoreType.DMA,
    ],
)(_patchify_body)


def _sc_scatter_body(sidx_hbm, scoef_hbm, part_hbm, idx_v, coef_v, acc_v, out_v,
                     sem):
    wid = lax.axis_index("s") * 2 + lax.axis_index("c")
    n = wid // 4
    qb = wid % 4
    lane = lax.iota(jnp.int32, 16)
    bases = [(((16 * k + lane) % MP) // N_POINTS) * LQ + (lane % N_POINTS) * (N_HEADS * LQ)
             for k in range(8)]
    zero16 = jnp.zeros((16,), jnp.float32)

    cp0 = pltpu.async_copy(sidx_hbm.at[n, pl.ds(qb * 256, 256)], idx_v, sem)
    cp1 = pltpu.async_copy(scoef_hbm.at[n, pl.ds(qb * 256, 256)], coef_v, sem)

    def zbody(i, carry):
        acc_v[pl.ds(i * 16, 16)] = zero16
        return carry
    lax.fori_loop(0, 2048, zbody, 0)
    cp0.wait()
    cp1.wait()

    def sbody(i, carry):
        for k in range(8):
            iv = idx_v[i, pl.ds(k * 16, 16)] + bases[k]
            plsc.addupdate_scatter(acc_v, [iv], coef_v[i, pl.ds(k * 16, 16)])
        return carry
    lax.fori_loop(0, 256, sbody, 0)

    def rbody(j, carry):
        h = j // 64
        g = (j % 64) // 8
        l16 = j % 8
        s0 = acc_v[pl.ds(j * 16, 16)] + acc_v[pl.ds(8192 + j * 16, 16)]
        s1 = acc_v[pl.ds(16384 + j * 16, 16)] + acc_v[pl.ds(24576 + j * 16, 16)]
        out_v[g * 8 + h, pl.ds(l16 * 16, 16)] = s0 + s1
        return carry
    lax.fori_loop(0, 512, rbody, 0)
    pltpu.sync_copy(out_v, part_hbm.at[pl.ds(wid * 64, 64)])


_sc_scatter = functools.partial(
    pl.kernel,
    out_type=jax.ShapeDtypeStruct((2048, 128), jnp.float32),
    mesh=plsc.VectorSubcoreMesh(core_axis_name="c", subcore_axis_name="s",
                                num_cores=2, num_subcores=16),
    compiler_params=pltpu.CompilerParams(use_tc_tiling_on_sc=False,
                                         needs_layout_passes=False),
    scratch_types=[
        pltpu.VMEM((256, 128), jnp.int32),
        pltpu.VMEM((256, 128), jnp.float32),
        pltpu.VMEM((4 * N_HEADS * LQ,), jnp.float32),
        pltpu.VMEM((64, 128), jnp.float32),
        pltpu.SemaphoreType.DMA,
    ],
)(_sc_scatter_body)


def kernel(x, embed_w, embed_b, so_w, so_b, aw_w, aw_b, vp_w, vp_b,
           op_w, op_b, proj_w, proj_b):
    B, T, C, H, W = x.shape
    N = B * T
    kdim = C * PATCH * PATCH

    x6 = x.reshape(N, C, HF, PATCH, HF, PATCH)
    patches = _sc_patchify(x6).reshape(N, 6, LQ, 128)
    we = embed_w.reshape(D_MODEL, 6, 128)
    so_w2 = so_w.reshape(N_HEADS, N_POINTS, 2, D_MODEL).transpose(2, 0, 1, 3).reshape(2 * MP, D_MODEL)
    so_b2 = so_b.reshape(N_HEADS, N_POINTS, 2).transpose(2, 0, 1).reshape(1, 2 * MP)

    rep = lambda *_: (0, 0)
    rep3 = lambda *_: (0, 0, 0)
    val, sidx, scoef = pl.pallas_call(
        _embed_body,
        grid=(N,),
        in_specs=[
            pl.BlockSpec((1, 6, LQ, 128), lambda n: (n, 0, 0, 0)),
            pl.BlockSpec((D_MODEL, 6, 128), rep3),
            pl.BlockSpec((1, D_MODEL), rep),
            pl.BlockSpec((D_MODEL, D_MODEL), rep),
            pl.BlockSpec((1, D_MODEL), rep),
            pl.BlockSpec((2 * MP, D_MODEL), rep),
            pl.BlockSpec((1, 2 * MP), rep),
            pl.BlockSpec((MP, D_MODEL), rep),
            pl.BlockSpec((1, MP), rep),
        ],
        out_specs=[
            pl.BlockSpec((1, LQ, D_MODEL), lambda n: (n, 0, 0)),
            pl.BlockSpec((1, LQ, 128), lambda n: (n, 0, 0)),
            pl.BlockSpec((1, LQ, 128), lambda n: (n, 0, 0)),
        ],
        out_shape=[
            jax.ShapeDtypeStruct((N, LQ, D_MODEL), jnp.float32),
            jax.ShapeDtypeStruct((N, LQ, 128), jnp.int32),
            jax.ShapeDtypeStruct((N, LQ, 128), jnp.float32),
        ],
    )(patches, we.astype(jnp.bfloat16), embed_b.reshape(1, D_MODEL),
      vp_w.astype(jnp.bfloat16), vp_b.reshape(1, D_MODEL),
      so_w2.astype(jnp.bfloat16), so_b2,
      aw_w.astype(jnp.bfloat16), aw_b.reshape(1, MP))

    coeff = _sc_scatter(sidx, scoef).reshape(N, 4, 64, 128)

    out = pl.pallas_call(
        _contract_body,
        grid=(N,),
        in_specs=[
            pl.BlockSpec((1, 4, 64, 128), lambda n: (n, 0, 0, 0)),
            pl.BlockSpec((1, LQ, D_MODEL), lambda n: (n, 0, 0)),
            pl.BlockSpec((D_MODEL, D_MODEL), rep),
            pl.BlockSpec((1, D_MODEL), rep),
            pl.BlockSpec((D_MODEL, D_MODEL), rep),
            pl.BlockSpec((1, D_MODEL), rep),
        ],
        out_specs=pl.BlockSpec((1, 1, D_MODEL), lambda n: (n, 0, 0)),
        out_shape=jax.ShapeDtypeStruct((N, 1, D_MODEL), jnp.float32),
    )(coeff, val, op_w, op_b.reshape(1, D_MODEL), proj_w, proj_b.reshape(1, D_MODEL))

    return out.reshape(B, T, D_MODEL)


# speedup vs baseline: 1.0120x; 1.0120x over previous
import functools

import jax
import jax.numpy as jnp
from jax import lax
from jax.experimental import pallas as pl
from jax.experimental.pallas import tpu as pltpu
from jax.experimental.pallas import tpu_sc as plsc

D_MODEL = 384
N_HEADS = 8
N_POINTS = 4
PATCH = 16
C_IN = 3
HF = 32
LQ = HF * HF
DH = D_MODEL // N_HEADS
MP = N_HEADS * N_POINTS


def _patchify_body(x_hbm, p_hbm, pbuf, sem):
    wid = lax.axis_index("s") * 2 + lax.axis_index("c")
    n = wid // 4
    qb = wid % 4

    def round_(rr, carry):
        qy0 = qb * 8 + rr * 4
        copies = []
        for j in range(48):
            c, py = divmod(j, PATCH)
            i, sub = divmod(j, 8)
            copies.append(pltpu.async_copy(
                x_hbm.at[n, c, pl.ds(qy0, 4), py, :, :],
                pbuf.at[i, :, :, pl.ds(sub * 16, 16)], sem))
        for cp in copies:
            cp.wait()
        pltpu.sync_copy(pbuf, p_hbm.at[n, :, qb * 2 + rr])
        return carry
    lax.fori_loop(0, 2, round_, 0)


def _dott(a, b):
    return lax.dot_general(a, b, (((1,), (1,)), ((), ())),
                           preferred_element_type=jnp.float32)


def _embed_body(p_ref, we_ref, be_ref, wv_ref, bv_ref, wso_ref, bso_ref,
                waw_ref, baw_ref, val_ref, sidx_ref, scoef_ref):
    p6 = p_ref[0].astype(jnp.bfloat16)
    feat = be_ref[...]
    for i in range(6):
        feat = feat + _dott(p6[i], we_ref[:, i, :])
    fb = feat.astype(jnp.bfloat16)
    val_ref[0] = _dott(fb, wv_ref[...]) + bv_ref[...]
    so = _dott(fb, wso_ref[...]) + bso_ref[...]
    awl = _dott(fb, waw_ref[...]) + baw_ref[...]

    rowmax = jnp.max(awl, axis=1, keepdims=True)
    e = jnp.exp(awl - rowmax)
    gi = lax.broadcasted_iota(jnp.int32, (MP, MP), 0) // N_POINTS
    gj = lax.broadcasted_iota(jnp.int32, (MP, MP), 1) // N_POINTS
    smat = (gi == gj).astype(jnp.float32)
    aw = e / jnp.dot(e, smat, preferred_element_type=jnp.float32)

    q = lax.broadcasted_iota(jnp.int32, (LQ, 1), 0)
    qx = (q % HF).astype(jnp.float32)
    qy = (q // HF).astype(jnp.float32)
    sc = jnp.float32(HF / (HF - 1.0))
    gx = qx * sc - 0.5 + so[:, :MP]
    gy = qy * sc - 0.5 + so[:, MP:]

    x0 = jnp.floor(gx)
    y0 = jnp.floor(gy)
    wx1 = gx - x0
    wx0 = 1.0 - wx1
    wy1 = gy - y0
    wy0 = 1.0 - wy1
    f32 = jnp.float32
    vx0 = ((x0 >= 0) & (x0 <= HF - 1)).astype(f32)
    vx1 = ((x0 >= -1) & (x0 <= HF - 2)).astype(f32)
    vy0 = ((y0 >= 0) & (y0 <= HF - 1)).astype(f32)
    vy1 = ((y0 >= -1) & (y0 <= HF - 2)).astype(f32)
    ax0 = wx0 * vx0
    ax1 = wx1 * vx1
    t0 = aw * (wy0 * vy0)
    t1 = aw * (wy1 * vy1)
    x0c = jnp.clip(x0, 0.0, HF - 1.0).astype(jnp.int32)
    x1c = jnp.clip(x0 + 1.0, 0.0, HF - 1.0).astype(jnp.int32)
    y0c = jnp.clip(y0, 0.0, HF - 1.0).astype(jnp.int32)
    y1c = jnp.clip(y0 + 1.0, 0.0, HF - 1.0).astype(jnp.int32)
    x128 = jnp.concatenate([x0c, x1c, x0c, x1c], axis=1)
    y128 = jnp.concatenate([y0c, y0c, y1c, y1c], axis=1)
    t128 = jnp.concatenate([t0, t0, t1, t1], axis=1)
    ax128 = jnp.concatenate([ax0, ax1, ax0, ax1], axis=1)
    sidx_ref[0] = y128 * HF + x128
    scoef_ref[0] = t128 * ax128


def _contract_body(coef_ref, val_ref, opw_ref, opb_ref, prw_ref, prb_ref, out_ref):
    c4 = coef_ref[0]
    csum = (c4[0] + c4[1]) + (c4[2] + c4[3])
    v3 = val_ref[0].reshape(N_HEADS, 128, D_MODEL)
    r = jnp.zeros((N_HEADS, D_MODEL), jnp.float32)
    for g in range(8):
        r = r + jnp.dot(csum[g * 8:(g + 1) * 8], v3[g],
                        preferred_element_type=jnp.float32)
    col = lax.broadcasted_iota(jnp.int32, (N_HEADS, D_MODEL), 1) // DH
    row = lax.broadcasted_iota(jnp.int32, (N_HEADS, D_MODEL), 0)
    mask = (col == row).astype(jnp.float32)
    osum = jnp.sum(r * mask, axis=0, keepdims=True) * jnp.float32(1.0 / LQ)
    h = _dott(osum, opw_ref[...]) + opb_ref[...]
    out_ref[0] = _dott(h, prw_ref[...]) + prb_ref[...]


_sc_patchify = functools.partial(
    pl.kernel,
    out_type=jax.ShapeDtypeStruct((8, 6, 8, 4, HF, 128), jnp.float32),
    mesh=plsc.VectorSubcoreMesh(core_axis_name="c", subcore_axis_name="s",
                                num_cores=2, num_subcores=16),
    compiler_params=pltpu.CompilerParams(use_tc_tiling_on_sc=False,
                                         needs_layout_passes=False),
    scratch_types=[
        pltpu.VMEM((6, 4, HF, 128), jnp.float32),
        pltpu.SemaphoreType.DMA,
    ],
)(_patchify_body)


def _sc_scatter_body(sidx_hbm, scoef_hbm, part_hbm, idx_v, coef_v, acc_v, out_v,
                     sem):
    wid = lax.axis_index("s") * 2 + lax.axis_index("c")
    n = wid // 4
    qb = wid % 4
    lane = lax.iota(jnp.int32, 16)
    bases = [(((16 * k + lane) % MP) // N_POINTS) * LQ + (lane % N_POINTS) * (N_HEADS * LQ)
             for k in range(8)]
    zero16 = jnp.zeros((16,), jnp.float32)

    cp0 = pltpu.async_copy(sidx_hbm.at[n, pl.ds(qb * 256, 256)], idx_v, sem)
    cp1 = pltpu.async_copy(scoef_hbm.at[n, pl.ds(qb * 256, 256)], coef_v, sem)

    def zbody(i, carry):
        acc_v[pl.ds(i * 16, 16)] = zero16
        return carry
    lax.fori_loop(0, 2048, zbody, 0)
    cp0.wait()
    cp1.wait()

    def sbody(i, carry):
        for k in range(8):
            iv = idx_v[i, pl.ds(k * 16, 16)] + bases[k]
            plsc.addupdate_scatter(acc_v, [iv], coef_v[i, pl.ds(k * 16, 16)])
        return carry
    lax.fori_loop(0, 256, sbody, 0)

    def rbody(j, carry):
        h = j // 64
        g = (j % 64) // 8
        l16 = j % 8
        s0 = acc_v[pl.ds(j * 16, 16)] + acc_v[pl.ds(8192 + j * 16, 16)]
        s1 = acc_v[pl.ds(16384 + j * 16, 16)] + acc_v[pl.ds(24576 + j * 16, 16)]
        out_v[g * 8 + h, pl.ds(l16 * 16, 16)] = s0 + s1
        return carry
    lax.fori_loop(0, 512, rbody, 0)
    pltpu.sync_copy(out_v, part_hbm.at[pl.ds(wid * 64, 64)])


_sc_scatter = functools.partial(
    pl.kernel,
    out_type=jax.ShapeDtypeStruct((2048, 128), jnp.float32),
    mesh=plsc.VectorSubcoreMesh(core_axis_name="c", subcore_axis_name="s",
                                num_cores=2, num_subcores=16),
    compiler_params=pltpu.CompilerParams(use_tc_tiling_on_sc=False,
                                         needs_layout_passes=False),
    scratch_types=[
        pltpu.VMEM((256, 128), jnp.int32),
        pltpu.VMEM((256, 128), jnp.float32),
        pltpu.VMEM((4 * N_HEADS * LQ,), jnp.float32),
        pltpu.VMEM((64, 128), jnp.float32),
        pltpu.SemaphoreType.DMA,
    ],
)(_sc_scatter_body)


def kernel(x, embed_w, embed_b, so_w, so_b, aw_w, aw_b, vp_w, vp_b,
           op_w, op_b, proj_w, proj_b):
    B, T, C, H, W = x.shape
    N = B * T
    kdim = C * PATCH * PATCH

    x6 = x.reshape(N, C, HF, PATCH, HF, PATCH)
    patches = _sc_patchify(x6).reshape(N, 6, LQ, 128)
    we = embed_w.reshape(D_MODEL, 6, 128)
    so_w2 = so_w.reshape(N_HEADS, N_POINTS, 2, D_MODEL).transpose(2, 0, 1, 3).reshape(2 * MP, D_MODEL)
    so_b2 = so_b.reshape(N_HEADS, N_POINTS, 2).transpose(2, 0, 1).reshape(1, 2 * MP)

    rep = lambda *_: (0, 0)
    rep3 = lambda *_: (0, 0, 0)
    val, sidx, scoef = pl.pallas_call(
        _embed_body,
        grid=(N,),
        in_specs=[
            pl.BlockSpec((1, 6, LQ, 128), lambda n: (n, 0, 0, 0)),
            pl.BlockSpec((D_MODEL, 6, 128), rep3),
            pl.BlockSpec((1, D_MODEL), rep),
            pl.BlockSpec((D_MODEL, D_MODEL), rep),
            pl.BlockSpec((1, D_MODEL), rep),
            pl.BlockSpec((2 * MP, D_MODEL), rep),
            pl.BlockSpec((1, 2 * MP), rep),
            pl.BlockSpec((MP, D_MODEL), rep),
            pl.BlockSpec((1, MP), rep),
        ],
        out_specs=[
            pl.BlockSpec((1, LQ, D_MODEL), lambda n: (n, 0, 0)),
            pl.BlockSpec((1, LQ, 128), lambda n: (n, 0, 0)),
            pl.BlockSpec((1, LQ, 128), lambda n: (n, 0, 0)),
        ],
        out_shape=[
            jax.ShapeDtypeStruct((N, LQ, D_MODEL), jnp.float32),
            jax.ShapeDtypeStruct((N, LQ, 128), jnp.int32),
            jax.ShapeDtypeStruct((N, LQ, 128), jnp.float32),
        ],
    )(patches, we.astype(jnp.bfloat16), embed_b.reshape(1, D_MODEL),
      vp_w.astype(jnp.bfloat16), vp_b.reshape(1, D_MODEL),
      so_w2.astype(jnp.bfloat16), so_b2,
      aw_w.astype(jnp.bfloat16), aw_b.reshape(1, MP))

    coeff = _sc_scatter(sidx, scoef).reshape(N, 4, 64, 128)

    out = pl.pallas_call(
        _contract_body,
        grid=(N,),
        in_specs=[
            pl.BlockSpec((1, 4, 64, 128), lambda n: (n, 0, 0, 0)),
            pl.BlockSpec((1, LQ, D_MODEL), lambda n: (n, 0, 0)),
            pl.BlockSpec((D_MODEL, D_MODEL), rep),
            pl.BlockSpec((1, D_MODEL), rep),
            pl.BlockSpec((D_MODEL, D_MODEL), rep),
            pl.BlockSpec((1, D_MODEL), rep),
        ],
        out_specs=pl.BlockSpec((1, 1, D_MODEL), lambda n: (n, 0, 0)),
        out_shape=jax.ShapeDtypeStruct((N, 1, D_MODEL), jnp.float32),
    )(coeff, val, op_w, op_b.reshape(1, D_MODEL), proj_w, proj_b.reshape(1, D_MODEL))

    return out.reshape(B, T, D_MODEL)


# speedup vs baseline: 1.0209x; 1.0088x over previous
import functools

import jax
import jax.numpy as jnp
from jax import lax
from jax.experimental import pallas as pl
from jax.experimental.pallas import tpu as pltpu
from jax.experimental.pallas import tpu_sc as plsc

D_MODEL = 384
N_HEADS = 8
N_POINTS = 4
PATCH = 16
C_IN = 3
HF = 32
LQ = HF * HF
DH = D_MODEL // N_HEADS
MP = N_HEADS * N_POINTS


def _patchify_body(x_hbm, p_hbm, pbuf, sem):
    wid = lax.axis_index("s") * 2 + lax.axis_index("c")
    n = wid // 4
    qb = wid % 4

    def round_(rr, carry):
        qy0 = qb * 8 + rr * 4
        copies = []
        for j in range(48):
            c, py = divmod(j, PATCH)
            i, sub = divmod(j, 8)
            copies.append(pltpu.async_copy(
                x_hbm.at[n, c, pl.ds(qy0, 4), py, :, :],
                pbuf.at[i, :, :, pl.ds(sub * 16, 16)], sem))
        for cp in copies:
            cp.wait()
        pltpu.sync_copy(pbuf, p_hbm.at[n, :, qb * 2 + rr])
        return carry
    lax.fori_loop(0, 2, round_, 0)


def _embed_body(p_ref, we_ref, be_ref, wv_ref, bv_ref, wso_ref, bso_ref,
                waw_ref, baw_ref, val_ref, sidx_ref, scoef_ref):
    p6 = p_ref[0].astype(jnp.bfloat16)
    feat = be_ref[...]
    for i in range(6):
        feat = feat + jnp.dot(p6[i], we_ref[i], preferred_element_type=jnp.float32)
    fb = feat.astype(jnp.bfloat16)
    val_ref[0] = jnp.dot(fb, wv_ref[...], preferred_element_type=jnp.float32) + bv_ref[...]
    so = jnp.dot(fb, wso_ref[...], preferred_element_type=jnp.float32) + bso_ref[...]
    awl = jnp.dot(fb, waw_ref[...], preferred_element_type=jnp.float32) + baw_ref[...]

    rowmax = jnp.max(awl, axis=1, keepdims=True)
    e = jnp.exp(awl - rowmax)
    gi = lax.broadcasted_iota(jnp.int32, (MP, MP), 0) // N_POINTS
    gj = lax.broadcasted_iota(jnp.int32, (MP, MP), 1) // N_POINTS
    smat = (gi == gj).astype(jnp.float32)
    aw = e / jnp.dot(e, smat, preferred_element_type=jnp.float32)

    q = lax.broadcasted_iota(jnp.int32, (LQ, 1), 0)
    qx = (q % HF).astype(jnp.float32)
    qy = (q // HF).astype(jnp.float32)
    sc = jnp.float32(HF / (HF - 1.0))
    gx = qx * sc - 0.5 + so[:, :MP]
    gy = qy * sc - 0.5 + so[:, MP:]

    x0 = jnp.floor(gx)
    y0 = jnp.floor(gy)
    wx1 = gx - x0
    wx0 = 1.0 - wx1
    wy1 = gy - y0
    wy0 = 1.0 - wy1
    f32 = jnp.float32
    vx0 = ((x0 >= 0) & (x0 <= HF - 1)).astype(f32)
    vx1 = ((x0 >= -1) & (x0 <= HF - 2)).astype(f32)
    vy0 = ((y0 >= 0) & (y0 <= HF - 1)).astype(f32)
    vy1 = ((y0 >= -1) & (y0 <= HF - 2)).astype(f32)
    ax0 = wx0 * vx0
    ax1 = wx1 * vx1
    t0 = aw * (wy0 * vy0)
    t1 = aw * (wy1 * vy1)
    x0c = jnp.clip(x0, 0.0, HF - 1.0).astype(jnp.int32)
    x1c = jnp.clip(x0 + 1.0, 0.0, HF - 1.0).astype(jnp.int32)
    y0c = jnp.clip(y0, 0.0, HF - 1.0).astype(jnp.int32)
    y1c = jnp.clip(y0 + 1.0, 0.0, HF - 1.0).astype(jnp.int32)
    x128 = jnp.concatenate([x0c, x1c, x0c, x1c], axis=1)
    y128 = jnp.concatenate([y0c, y0c, y1c, y1c], axis=1)
    t128 = jnp.concatenate([t0, t0, t1, t1], axis=1)
    ax128 = jnp.concatenate([ax0, ax1, ax0, ax1], axis=1)
    sidx_ref[0] = y128 * HF + x128
    scoef_ref[0] = t128 * ax128


def _contract_body(coef_ref, val_ref, opw_ref, opb_ref, prw_ref, prb_ref, out_ref):
    c4 = coef_ref[0]
    csum = (c4[0] + c4[1]) + (c4[2] + c4[3])
    v3 = val_ref[0].reshape(N_HEADS, 128, D_MODEL)
    r = jnp.zeros((N_HEADS, D_MODEL), jnp.float32)
    for g in range(8):
        r = r + jnp.dot(csum[g * 8:(g + 1) * 8], v3[g],
                        preferred_element_type=jnp.float32)
    col = lax.broadcasted_iota(jnp.int32, (N_HEADS, D_MODEL), 1) // DH
    row = lax.broadcasted_iota(jnp.int32, (N_HEADS, D_MODEL), 0)
    mask = (col == row).astype(jnp.float32)
    osum = jnp.sum(r * mask, axis=0, keepdims=True) * jnp.float32(1.0 / LQ)
    h = jnp.dot(osum, opw_ref[...], preferred_element_type=jnp.float32) + opb_ref[...]
    out_ref[0] = jnp.dot(h, prw_ref[...], preferred_element_type=jnp.float32) + prb_ref[...]


_sc_patchify = functools.partial(
    pl.kernel,
    out_type=jax.ShapeDtypeStruct((8, 6, 8, 4, HF, 128), jnp.float32),
    mesh=plsc.VectorSubcoreMesh(core_axis_name="c", subcore_axis_name="s",
                                num_cores=2, num_subcores=16),
    compiler_params=pltpu.CompilerParams(use_tc_tiling_on_sc=False,
                                         needs_layout_passes=False),
    scratch_types=[
        pltpu.VMEM((6, 4, HF, 128), jnp.float32),
        pltpu.SemaphoreType.DMA,
    ],
)(_patchify_body)


def _sc_scatter_body(sidx_hbm, scoef_hbm, part_hbm, idx_v, coef_v, acc_v, out_v,
                     sem):
    wid = lax.axis_index("s") * 2 + lax.axis_index("c")
    n = wid // 4
    qb = wid % 4
    lane = lax.iota(jnp.int32, 16)
    bases = [(((16 * k + lane) % MP) // N_POINTS) * LQ + (lane % N_POINTS) * (N_HEADS * LQ)
             for k in range(8)]
    zero16 = jnp.zeros((16,), jnp.float32)

    cp0 = pltpu.async_copy(sidx_hbm.at[n, pl.ds(qb * 256, 256)], idx_v, sem)
    cp1 = pltpu.async_copy(scoef_hbm.at[n, pl.ds(qb * 256, 256)], coef_v, sem)

    def zbody(i, carry):
        acc_v[pl.ds(i * 16, 16)] = zero16
        return carry
    lax.fori_loop(0, 2048, zbody, 0)
    cp0.wait()
    cp1.wait()

    def sbody(i, carry):
        for k in range(8):
            iv = idx_v[i, pl.ds(k * 16, 16)] + bases[k]
            plsc.addupdate_scatter(acc_v, [iv], coef_v[i, pl.ds(k * 16, 16)])
        return carry
    lax.fori_loop(0, 256, sbody, 0)

    def rbody(j, carry):
        h = j // 64
        g = (j % 64) // 8
        l16 = j % 8
        s0 = acc_v[pl.ds(j * 16, 16)] + acc_v[pl.ds(8192 + j * 16, 16)]
        s1 = acc_v[pl.ds(16384 + j * 16, 16)] + acc_v[pl.ds(24576 + j * 16, 16)]
        out_v[g * 8 + h, pl.ds(l16 * 16, 16)] = s0 + s1
        return carry
    lax.fori_loop(0, 512, rbody, 0)
    pltpu.sync_copy(out_v, part_hbm.at[pl.ds(wid * 64, 64)])


_sc_scatter = functools.partial(
    pl.kernel,
    out_type=jax.ShapeDtypeStruct((2048, 128), jnp.float32),
    mesh=plsc.VectorSubcoreMesh(core_axis_name="c", subcore_axis_name="s",
                                num_cores=2, num_subcores=16),
    compiler_params=pltpu.CompilerParams(use_tc_tiling_on_sc=False,
                                         needs_layout_passes=False),
    scratch_types=[
        pltpu.VMEM((256, 128), jnp.int32),
        pltpu.VMEM((256, 128), jnp.float32),
        pltpu.VMEM((4 * N_HEADS * LQ,), jnp.float32),
        pltpu.VMEM((64, 128), jnp.float32),
        pltpu.SemaphoreType.DMA,
    ],
)(_sc_scatter_body)


def kernel(x, embed_w, embed_b, so_w, so_b, aw_w, aw_b, vp_w, vp_b,
           op_w, op_b, proj_w, proj_b):
    B, T, C, H, W = x.shape
    N = B * T
    kdim = C * PATCH * PATCH

    x6 = x.reshape(N, C, HF, PATCH, HF, PATCH)
    patches = _sc_patchify(x6).reshape(N, 6, LQ, 128)
    we = embed_w.transpose(1, 2, 3, 0).reshape(6, 128, D_MODEL)
    so_w2 = so_w.reshape(N_HEADS, N_POINTS, 2, D_MODEL).transpose(2, 0, 1, 3).reshape(2 * MP, D_MODEL)
    so_b2 = so_b.reshape(N_HEADS, N_POINTS, 2).transpose(2, 0, 1).reshape(1, 2 * MP)

    rep = lambda *_: (0, 0)
    rep3 = lambda *_: (0, 0, 0)
    val, sidx, scoef = pl.pallas_call(
        _embed_body,
        grid=(N,),
        in_specs=[
            pl.BlockSpec((1, 6, LQ, 128), lambda n: (n, 0, 0, 0)),
            pl.BlockSpec((6, 128, D_MODEL), rep3),
            pl.BlockSpec((1, D_MODEL), rep),
            pl.BlockSpec((D_MODEL, D_MODEL), rep),
            pl.BlockSpec((1, D_MODEL), rep),
            pl.BlockSpec((D_MODEL, 2 * MP), rep),
            pl.BlockSpec((1, 2 * MP), rep),
            pl.BlockSpec((D_MODEL, MP), rep),
            pl.BlockSpec((1, MP), rep),
        ],
        out_specs=[
            pl.BlockSpec((1, LQ, D_MODEL), lambda n: (n, 0, 0)),
            pl.BlockSpec((1, LQ, 128), lambda n: (n, 0, 0)),
            pl.BlockSpec((1, LQ, 128), lambda n: (n, 0, 0)),
        ],
        out_shape=[
            jax.ShapeDtypeStruct((N, LQ, D_MODEL), jnp.float32),
            jax.ShapeDtypeStruct((N, LQ, 128), jnp.int32),
            jax.ShapeDtypeStruct((N, LQ, 128), jnp.float32),
        ],
    )(patches, we.astype(jnp.bfloat16), embed_b.reshape(1, D_MODEL),
      vp_w.T.astype(jnp.bfloat16), vp_b.reshape(1, D_MODEL),
      so_w2.T.astype(jnp.bfloat16), so_b2,
      aw_w.T.astype(jnp.bfloat16), aw_b.reshape(1, MP))

    coeff = _sc_scatter(sidx, scoef).reshape(N, 4, 64, 128)

    out = pl.pallas_call(
        _contract_body,
        grid=(N,),
        in_specs=[
            pl.BlockSpec((1, 4, 64, 128), lambda n: (n, 0, 0, 0)),
            pl.BlockSpec((1, LQ, D_MODEL), lambda n: (n, 0, 0)),
            pl.BlockSpec((D_MODEL, D_MODEL), rep),
            pl.BlockSpec((1, D_MODEL), rep),
            pl.BlockSpec((D_MODEL, D_MODEL), rep),
            pl.BlockSpec((1, D_MODEL), rep),
        ],
        out_specs=pl.BlockSpec((1, 1, D_MODEL), lambda n: (n, 0, 0)),
        out_shape=jax.ShapeDtypeStruct((N, 1, D_MODEL), jnp.float32),
    )(coeff, val, op_w.T, op_b.reshape(1, D_MODEL), proj_w.T, proj_b.reshape(1, D_MODEL))

    return out.reshape(B, T, D_MODEL)
